# baseline (speedup 1.0000x reference)
import functools

import jax
import jax.numpy as jnp
from jax import lax
from jax.experimental import pallas as pl
from jax.experimental.pallas import tpu as pltpu
from jax.experimental.pallas import tpu_sc as plsc

NUM_E = 1_000_000
EMB = 16
B0, L0 = 16384, 20
N = B0 * L0
NC, NS = 2, 16
NW = NC * NS
TOK_W = N // NW
CH = 512
SUB = CH // 128
NCHUNK = TOK_W // CH
NROW = N // 128
F_CH = 5120
F_CROWS = F_CH // L0
F_OROWS = F_CH // 128


def _fmt_body(u_h, p_h, n_h, m_h, o_u, o_p, o_n, o_m, iri, irf, bi, bf):
    wid = lax.axis_index("s") * NC + lax.axis_index("c")
    lane = lax.iota(jnp.int32, 16)

    def one(src_h, dst_h, raw, buf, r20, rowb):
        pltpu.sync_copy(src_h.at[pl.ds(r20, F_CROWS)], raw)

        def flat(g, carry2):
            tokl = g * 16 + lane
            rl = tokl // L0
            cl = tokl - rl * L0
            buf[g // 8, pl.ds((g % 8) * 16, 16)] = plsc.load_gather(
                raw, [rl, cl])
            return carry2

        lax.fori_loop(0, F_CH // 16, flat, 0)
        pltpu.sync_copy(buf, dst_h.at[pl.ds(rowb, F_OROWS)])

    def chunk(c, carry):
        r20 = pl.multiple_of(wid * (TOK_W // L0) + c * F_CROWS, 8)
        rowb = pl.multiple_of(wid * (TOK_W // 128) + c * F_OROWS, 8)
        one(u_h, o_u, iri, bi, r20, rowb)
        one(p_h, o_p, iri, bi, r20, rowb)
        one(n_h, o_n, iri, bi, r20, rowb)
        one(m_h, o_m, irf, bf, r20, rowb)
        return carry

    lax.fori_loop(0, TOK_W // F_CH, chunk, 0)


def _make_fmt():
    mesh = plsc.VectorSubcoreMesh(core_axis_name="c", subcore_axis_name="s",
                                  num_cores=NC, num_subcores=NS)
    ivec = jax.ShapeDtypeStruct((NROW, 128), jnp.int32)
    fvec = jax.ShapeDtypeStruct((NROW, 128), jnp.float32)
    return pl.kernel(
        _fmt_body,
        out_type=(ivec, ivec, ivec, fvec),
        mesh=mesh,
        compiler_params=pltpu.CompilerParams(use_tc_tiling_on_sc=True,
                                             needs_layout_passes=False),
        scratch_types=[
            pltpu.VMEM((F_CROWS, L0), jnp.int32),
            pltpu.VMEM((F_CROWS, L0), jnp.float32),
            pltpu.VMEM((F_OROWS, 128), jnp.int32),
            pltpu.VMEM((F_OROWS, 128), jnp.float32),
        ],
    )

INT_W = 0.1
POP_W = 0.1
DIS_PEN = 0.01
GAMMA = 0.02

def _gather_body(user_h, itemp_h, itemn_h, tabs_h, blen_h,
                 o_pi, o_ni, o_pp, o_np, o_popp, o_popn, o_squ, o_sqp, o_sqn,
                 idx_u0, idx_p0, idx_n0, r00, r10, r20, r30, r40, r50,
                 bp0, bn0,
                 idx_u1, idx_p1, idx_n1, r01, r11, r21, r31, r41, r51,
                 bp1, bn1,
                 b_pi, b_ni, b_pp, b_np, b_squ, b_sqp, b_sqn, sem0, sem1):
    wid = lax.axis_index("s") * NC + lax.axis_index("c")
    lane = lax.iota(jnp.int32, 16)
    uint_h = tabs_h.at[0]
    upop_h = tabs_h.at[1]
    iint_h = tabs_h.at[2]
    ipop_h = tabs_h.at[3]
    sets = ((idx_u0, idx_p0, idx_n0, r00, r10, r20, r30, r40, r50, bp0, bn0,
             sem0),
            (idx_u1, idx_p1, idx_n1, r01, r11, r21, r31, r41, r51, bp1, bn1,
             sem1))

    def descs(c, st, make):
        idx_u, idx_p, idx_n, r0, r1, r2, r3, r4, r5, bp, bn, sem = st
        fn = pltpu.make_async_copy if make else pltpu.async_copy
        cps = []
        for j in range(SUB):
            sl = pl.ds(j * 128, 128)
            cps.append(fn(uint_h.at[idx_u.at[j]], r0.at[sl], sem))
            cps.append(fn(upop_h.at[idx_u.at[j]], r1.at[sl], sem))
            cps.append(fn(iint_h.at[idx_p.at[j]], r2.at[sl], sem))
            cps.append(fn(ipop_h.at[idx_p.at[j]], r3.at[sl], sem))
            cps.append(fn(iint_h.at[idx_n.at[j]], r4.at[sl], sem))
            cps.append(fn(ipop_h.at[idx_n.at[j]], r5.at[sl], sem))
            cps.append(fn(blen_h.at[idx_p.at[j]], bp.at[j], sem))
            cps.append(fn(blen_h.at[idx_n.at[j]], bn.at[j], sem))
        return cps

    def load_and_fire(c, st):
        idx_u, idx_p, idx_n = st[0], st[1], st[2]
        rowb = wid * (TOK_W // 128) + c * SUB
        pltpu.sync_copy(user_h.at[pl.ds(rowb, SUB)], idx_u)
        pltpu.sync_copy(itemp_h.at[pl.ds(rowb, SUB)], idx_p)
        pltpu.sync_copy(itemn_h.at[pl.ds(rowb, SUB)], idx_n)
        descs(c, st, make=False)

    def work(c, st, nxt):
        for cp in descs(c, st, make=True):
            cp.wait()

        @pl.when(c + 1 < NCHUNK)
        def _():
            load_and_fire(c + 1, nxt)

        r0, r1, r2, r3, r4, r5, bp, bn = st[3:11]
        rowb = wid * (TOK_W // 128) + c * SUB

        def grp(g, carry2):
            tok = g * 16 + lane
            p_i = None
            for e in range(EMB):
                ce = jnp.full((16,), e, jnp.int32)
                u = plsc.load_gather(r0, [tok, ce])
                up_ = plsc.load_gather(r1, [tok, ce])
                pi = plsc.load_gather(r2, [tok, ce])
                pp = plsc.load_gather(r3, [tok, ce])
                ni = plsc.load_gather(r4, [tok, ce])
                npp = plsc.load_gather(r5, [tok, ce])
                du = u - up_
                dp = pi - pp
                dn = ni - npp
                if p_i is None:
                    p_i = u * pi
                    n_i = u * ni
                    p_p = up_ * pp
                    n_p = up_ * npp
                    s_u = du * du
                    s_p = dp * dp
                    s_n = dn * dn
                else:
                    p_i = p_i + u * pi
                    n_i = n_i + u * ni
                    p_p = p_p + up_ * pp
                    n_p = n_p + up_ * npp
                    s_u = s_u + du * du
                    s_p = s_p + dp * dp
                    s_n = s_n + dn * dn
            gi = g // 8
            gl = pl.ds((g % 8) * 16, 16)
            b_pi[gi, gl] = p_i
            b_ni[gi, gl] = n_i
            b_pp[gi, gl] = p_p
            b_np[gi, gl] = n_p
            b_squ[gi, gl] = s_u
            b_sqp[gi, gl] = s_p
            b_sqn[gi, gl] = s_n
            return carry2

        lax.fori_loop(0, CH // 16, grp, 0)
        dst = pl.ds(rowb, SUB)
        pltpu.sync_copy(b_pi, o_pi.at[dst])
        pltpu.sync_copy(b_ni, o_ni.at[dst])
        pltpu.sync_copy(b_pp, o_pp.at[dst])
        pltpu.sync_copy(b_np, o_np.at[dst])
        pltpu.sync_copy(b_squ, o_squ.at[dst])
        pltpu.sync_copy(b_sqp, o_sqp.at[dst])
        pltpu.sync_copy(b_sqn, o_sqn.at[dst])
        pltpu.sync_copy(bp, o_popp.at[dst])
        pltpu.sync_copy(bn, o_popn.at[dst])

    load_and_fire(0, sets[0])

    def chunk(c, carry):
        @pl.when(c % 2 == 0)
        def _():
            work(c, sets[0], sets[1])

        @pl.when(c % 2 == 1)
        def _():
            work(c, sets[1], sets[0])

        return carry

    lax.fori_loop(0, NCHUNK, chunk, 0)


def _make_gather():
    mesh = plsc.VectorSubcoreMesh(core_axis_name="c", subcore_axis_name="s",
                                  num_cores=NC, num_subcores=NS)
    vec = jax.ShapeDtypeStruct((NROW, 128), jnp.float32)
    return pl.kernel(
        _gather_body,
        out_type=(vec,) * 9,
        mesh=mesh,
        compiler_params=pltpu.CompilerParams(use_tc_tiling_on_sc=False,
                                             needs_layout_passes=False),
        scratch_types=(
            [pltpu.VMEM((SUB, 128), jnp.int32)] * 3
            + [pltpu.VMEM((CH, EMB), jnp.float32)] * 6
            + [pltpu.VMEM((SUB, 128), jnp.float32)] * 2
            + [pltpu.VMEM((SUB, 128), jnp.int32)] * 3
            + [pltpu.VMEM((CH, EMB), jnp.float32)] * 6
            + [pltpu.VMEM((SUB, 128), jnp.float32)] * 2
            + [pltpu.VMEM((SUB, 128), jnp.float32)] * 7
            + [pltpu.SemaphoreType.DMA, pltpu.SemaphoreType.DMA]
        ),
    )


TR = 256
GRID = NROW // TR


def _loss_body(pint, nint, ppop, npop, popp, popn, mkf, acc):
    i = pl.program_id(0)
    p_int = pint[...]
    n_int = nint[...]
    p_pop = ppop[...]
    n_pop = npop[...]
    mk = mkf[...]

    def logsig(x):
        return -(jnp.maximum(-x, 0.0) + jnp.log1p(jnp.exp(-jnp.abs(x))))

    t1 = mk * logsig(p_int - n_int)
    t2 = mk * logsig(n_pop - p_pop)
    ep = jnp.where(p_pop > 0, p_pop, jnp.exp(p_pop) - 1.0) + 1.0
    en = jnp.where(n_pop > 0, n_pop, jnp.exp(n_pop) - 1.0) + 1.0
    gp = jnp.exp(GAMMA * jnp.log(popp[...]))
    gn = jnp.exp(GAMMA * jnp.log(popn[...]))
    t3 = (1.0 - mk) * logsig(ep * gp - en * gn)
    t4 = logsig(p_int + p_pop - n_int - n_pop)
    part = jnp.stack([t1, t2, t3, t4])
    part = part.reshape(4, TR // 8, 8, 128).sum(axis=1)

    @pl.when(i == 0)
    def _():
        acc[...] = jnp.zeros_like(acc)

    acc[...] += part


def _make_loss():
    vec_spec = pl.BlockSpec((TR, 128), lambda i: (i, 0))
    acc_spec = pl.BlockSpec((4, 8, 128), lambda i: (0, 0, 0))
    return pl.pallas_call(
        _loss_body,
        grid=(GRID,),
        in_specs=[vec_spec] * 7,
        out_specs=acc_spec,
        out_shape=jax.ShapeDtypeStruct((4, 8, 128), jnp.float32),
    )


RED = 4096
WPT = NUM_E // NS
SROW = 32
TPT = N // NS
RPT = TPT // 128


def _dedup_body(idxp_h, idxn_h, idxu_h, sqp_h, sqn_h, squ_h, out_h,
                table, ibuf, sbuf, rbuf, obuf, ssem):
    cid = lax.axis_index("c")
    sid = lax.axis_index("s")

    def fill16(i, carry):
        rbuf[pl.ds(i * 16, 16)] = jnp.full((16,), -1.0, jnp.float32)
        return carry

    lax.fori_loop(0, RED // 16, fill16, 0)

    def memset(i, carry):
        off = pl.multiple_of(sid * WPT + i * RED, 8)
        pltpu.sync_copy(rbuf, table.at[pl.ds(off, RED)])
        return carry

    lax.fori_loop(0, WPT // RED, memset, 0)
    plsc.subcore_barrier()

    def scatter_from(idx_h, sq_h):
        def body(ci, carry):
            rb = sid * RPT + ci * SROW
            pltpu.sync_copy(idx_h.at[pl.ds(rb, SROW)], ibuf)
            pltpu.sync_copy(sq_h.at[pl.ds(rb, SROW)], sbuf)
            cps = [pltpu.async_copy(sbuf.at[r], table.at[ibuf.at[r]], ssem)
                   for r in range(SROW)]
            for cp in cps:
                cp.wait()
            return carry

        lax.fori_loop(0, RPT // SROW, body, 0)

    @pl.when(cid == 0)
    def _():
        scatter_from(idxp_h, sqp_h)
        scatter_from(idxn_h, sqn_h)

    @pl.when(cid == 1)
    def _():
        scatter_from(idxu_h, squ_h)

    plsc.subcore_barrier()

    def red_chunk(i, carry):
        off = pl.multiple_of(sid * WPT + i * RED, 8)
        pltpu.sync_copy(table.at[pl.ds(off, RED)], rbuf)

        def red16(v, c2):
            s2, n2 = c2
            for u in range(4):
                x = rbuf[pl.ds(v * 64 + u * 16, 16)]
                m = x >= 0.0
                s2 = s2 + jnp.where(m, x, 0.0)
                n2 = n2 + jnp.where(m, 1.0, 0.0)
            return (s2, n2)

        return lax.fori_loop(0, RED // 64, red16, carry)

    zero = jnp.zeros((16,), jnp.float32)
    ssum, scnt = lax.fori_loop(0, WPT // RED, red_chunk, (zero, zero))
    obuf[0, :] = ssum
    obuf[1, :] = scnt
    pltpu.sync_copy(obuf, out_h.at[cid, sid])


def _make_dedup():
    mesh = plsc.VectorSubcoreMesh(core_axis_name="c", subcore_axis_name="s",
                                  num_cores=NC, num_subcores=NS)
    return pl.kernel(
        _dedup_body,
        out_type=jax.ShapeDtypeStruct((NC, NS, 2, 16), jnp.float32),
        mesh=mesh,
        scratch_types=[
            pltpu.VMEM_SHARED((NUM_E,), jnp.float32),
            pltpu.VMEM((SROW, 128), jnp.int32),
            pltpu.VMEM((SROW, 128), jnp.float32),
            pltpu.VMEM((RED,), jnp.float32),
            pltpu.VMEM((2, 16), jnp.float32),
            pltpu.SemaphoreType.DMA,
        ],
    )


def kernel(user, item_p, item_n, mask, users_int_w, users_pop_w, items_int_w,
           items_pop_w, blen_pop):
    mkf_raw = mask.astype(jnp.float32)
    user2, itemp2, itemn2, mkf = _make_fmt()(user, item_p, item_n, mkf_raw)

    tabs = jnp.stack([users_int_w, users_pop_w, items_int_w, items_pop_w])
    (p_int, n_int, p_pop, n_pop, popp, popn, squ, sqp, sqn) = (
        _make_gather()(user2, itemp2, itemn2, tabs, blen_pop))

    acc = _make_loss()(p_int, n_int, p_pop, n_pop, popp, popn, mkf)

    dd = _make_dedup()(itemp2, itemn2, user2, sqp, sqn, squ)

    sums = jnp.sum(acc, axis=(1, 2))
    nf = jnp.float32(N)
    loss_int = -sums[0] / nf
    loss_pop = -sums[1] / nf - sums[2] / nf
    loss_total = -sums[3] / nf
    item_sum = jnp.sum(dd[0, :, 0, :])
    item_cnt = jnp.sum(dd[0, :, 1, :])
    user_sum = jnp.sum(dd[1, :, 0, :])
    user_cnt = jnp.sum(dd[1, :, 1, :])
    disc = item_sum / (item_cnt * EMB) + user_sum / (user_cnt * EMB)
    return (INT_W * loss_int + POP_W * loss_pop + loss_total
            - DIS_PEN * disc)


# speedup vs baseline: 1.2316x; 1.2316x over previous
import functools

import jax
import jax.numpy as jnp
from jax import lax
from jax.experimental import pallas as pl
from jax.experimental.pallas import tpu as pltpu
from jax.experimental.pallas import tpu_sc as plsc

NUM_E = 1_000_000
EMB = 16
B0, L0 = 16384, 20
N = B0 * L0
NC, NS = 2, 16
NW = NC * NS
TOK_W = N // NW
CH = 512
SUB = CH // 128
NCHUNK = TOK_W // CH
NROW = N // 128
F_CH = 5120
F_CROWS = F_CH // L0
F_OROWS = F_CH // 128


def _fmt_body(u_h, p_h, n_h, m_h, o_u, o_p, o_n, o_m, iri, irf, bi, bf):
    wid = lax.axis_index("s") * NC + lax.axis_index("c")
    lane = lax.iota(jnp.int32, 16)

    def one(src_h, dst_h, raw, buf, r20, rowb):
        pltpu.sync_copy(src_h.at[pl.ds(r20, F_CROWS)], raw)

        def flat(g, carry2):
            tokl = g * 16 + lane
            rl = tokl // L0
            cl = tokl - rl * L0
            buf[g // 8, pl.ds((g % 8) * 16, 16)] = plsc.load_gather(
                raw, [rl, cl])
            return carry2

        lax.fori_loop(0, F_CH // 16, flat, 0)
        pltpu.sync_copy(buf, dst_h.at[pl.ds(rowb, F_OROWS)])

    def chunk(c, carry):
        r20 = pl.multiple_of(wid * (TOK_W // L0) + c * F_CROWS, 8)
        rowb = pl.multiple_of(wid * (TOK_W // 128) + c * F_OROWS, 8)
        one(u_h, o_u, iri, bi, r20, rowb)
        one(p_h, o_p, iri, bi, r20, rowb)
        one(n_h, o_n, iri, bi, r20, rowb)
        one(m_h, o_m, irf, bf, r20, rowb)
        return carry

    lax.fori_loop(0, TOK_W // F_CH, chunk, 0)


def _make_fmt():
    mesh = plsc.VectorSubcoreMesh(core_axis_name="c", subcore_axis_name="s",
                                  num_cores=NC, num_subcores=NS)
    ivec = jax.ShapeDtypeStruct((NROW, 128), jnp.int32)
    fvec = jax.ShapeDtypeStruct((NROW, 128), jnp.float32)
    return pl.kernel(
        _fmt_body,
        out_type=(ivec, ivec, ivec, fvec),
        mesh=mesh,
        compiler_params=pltpu.CompilerParams(use_tc_tiling_on_sc=True,
                                             needs_layout_passes=False),
        scratch_types=[
            pltpu.VMEM((F_CROWS, L0), jnp.int32),
            pltpu.VMEM((F_CROWS, L0), jnp.float32),
            pltpu.VMEM((F_OROWS, 128), jnp.int32),
            pltpu.VMEM((F_OROWS, 128), jnp.float32),
        ],
    )

INT_W = 0.1
POP_W = 0.1
DIS_PEN = 0.01
GAMMA = 0.02

def _gather_body(user_h, itemp_h, itemn_h, uint_h, upop_h, iint_h,
                 ipop_h, blen_h,
                 o_pi, o_ni, o_pp, o_np, o_popp, o_popn, o_squ, o_sqp, o_sqn,
                 idx_u0, idx_p0, idx_n0, r00, r10, r20, r30, r40, r50,
                 bp0, bn0,
                 idx_u1, idx_p1, idx_n1, r01, r11, r21, r31, r41, r51,
                 bp1, bn1,
                 b_pi, b_ni, b_pp, b_np, b_squ, b_sqp, b_sqn, sem0, sem1):
    wid = lax.axis_index("s") * NC + lax.axis_index("c")
    lane = lax.iota(jnp.int32, 16)
    sets = ((idx_u0, idx_p0, idx_n0, r00, r10, r20, r30, r40, r50, bp0, bn0,
             sem0),
            (idx_u1, idx_p1, idx_n1, r01, r11, r21, r31, r41, r51, bp1, bn1,
             sem1))

    def descs(c, st, make):
        idx_u, idx_p, idx_n, r0, r1, r2, r3, r4, r5, bp, bn, sem = st
        fn = pltpu.make_async_copy if make else pltpu.async_copy
        cps = []
        for j in range(SUB):
            sl = pl.ds(j * 128, 128)
            cps.append(fn(uint_h.at[idx_u.at[j]], r0.at[sl], sem))
            cps.append(fn(upop_h.at[idx_u.at[j]], r1.at[sl], sem))
            cps.append(fn(iint_h.at[idx_p.at[j]], r2.at[sl], sem))
            cps.append(fn(ipop_h.at[idx_p.at[j]], r3.at[sl], sem))
            cps.append(fn(iint_h.at[idx_n.at[j]], r4.at[sl], sem))
            cps.append(fn(ipop_h.at[idx_n.at[j]], r5.at[sl], sem))
            cps.append(fn(blen_h.at[idx_p.at[j]], bp.at[j], sem))
            cps.append(fn(blen_h.at[idx_n.at[j]], bn.at[j], sem))
        return cps

    def load_and_fire(c, st):
        idx_u, idx_p, idx_n = st[0], st[1], st[2]
        rowb = wid * (TOK_W // 128) + c * SUB
        pltpu.sync_copy(user_h.at[pl.ds(rowb, SUB)], idx_u)
        pltpu.sync_copy(itemp_h.at[pl.ds(rowb, SUB)], idx_p)
        pltpu.sync_copy(itemn_h.at[pl.ds(rowb, SUB)], idx_n)
        descs(c, st, make=False)

    def work(c, st, nxt):
        for cp in descs(c, st, make=True):
            cp.wait()

        @pl.when(c + 1 < NCHUNK)
        def _():
            load_and_fire(c + 1, nxt)

        r0, r1, r2, r3, r4, r5, bp, bn = st[3:11]
        rowb = wid * (TOK_W // 128) + c * SUB

        def grp(g, carry2):
            tok = g * 16 + lane
            p_i = None
            for e in range(EMB):
                ce = jnp.full((16,), e, jnp.int32)
                u = plsc.load_gather(r0, [tok, ce])
                up_ = plsc.load_gather(r1, [tok, ce])
                pi = plsc.load_gather(r2, [tok, ce])
                pp = plsc.load_gather(r3, [tok, ce])
                ni = plsc.load_gather(r4, [tok, ce])
                npp = plsc.load_gather(r5, [tok, ce])
                du = u - up_
                dp = pi - pp
                dn = ni - npp
                if p_i is None:
                    p_i = u * pi
                    n_i = u * ni
                    p_p = up_ * pp
                    n_p = up_ * npp
                    s_u = du * du
                    s_p = dp * dp
                    s_n = dn * dn
                else:
                    p_i = p_i + u * pi
                    n_i = n_i + u * ni
                    p_p = p_p + up_ * pp
                    n_p = n_p + up_ * npp
                    s_u = s_u + du * du
                    s_p = s_p + dp * dp
                    s_n = s_n + dn * dn
            gi = g // 8
            gl = pl.ds((g % 8) * 16, 16)
            b_pi[gi, gl] = p_i
            b_ni[gi, gl] = n_i
            b_pp[gi, gl] = p_p
            b_np[gi, gl] = n_p
            b_squ[gi, gl] = s_u
            b_sqp[gi, gl] = s_p
            b_sqn[gi, gl] = s_n
            return carry2

        lax.fori_loop(0, CH // 16, grp, 0)
        dst = pl.ds(rowb, SUB)
        pltpu.sync_copy(b_pi, o_pi.at[dst])
        pltpu.sync_copy(b_ni, o_ni.at[dst])
        pltpu.sync_copy(b_pp, o_pp.at[dst])
        pltpu.sync_copy(b_np, o_np.at[dst])
        pltpu.sync_copy(b_squ, o_squ.at[dst])
        pltpu.sync_copy(b_sqp, o_sqp.at[dst])
        pltpu.sync_copy(b_sqn, o_sqn.at[dst])
        pltpu.sync_copy(bp, o_popp.at[dst])
        pltpu.sync_copy(bn, o_popn.at[dst])

    load_and_fire(0, sets[0])

    def chunk(c, carry):
        @pl.when(c % 2 == 0)
        def _():
            work(c, sets[0], sets[1])

        @pl.when(c % 2 == 1)
        def _():
            work(c, sets[1], sets[0])

        return carry

    lax.fori_loop(0, NCHUNK, chunk, 0)


def _make_gather():
    mesh = plsc.VectorSubcoreMesh(core_axis_name="c", subcore_axis_name="s",
                                  num_cores=NC, num_subcores=NS)
    vec = jax.ShapeDtypeStruct((NROW, 128), jnp.float32)
    return pl.kernel(
        _gather_body,
        out_type=(vec,) * 9,
        mesh=mesh,
        compiler_params=pltpu.CompilerParams(use_tc_tiling_on_sc=False,
                                             needs_layout_passes=False),
        scratch_types=(
            [pltpu.VMEM((SUB, 128), jnp.int32)] * 3
            + [pltpu.VMEM((CH, EMB), jnp.float32)] * 6
            + [pltpu.VMEM((SUB, 128), jnp.float32)] * 2
            + [pltpu.VMEM((SUB, 128), jnp.int32)] * 3
            + [pltpu.VMEM((CH, EMB), jnp.float32)] * 6
            + [pltpu.VMEM((SUB, 128), jnp.float32)] * 2
            + [pltpu.VMEM((SUB, 128), jnp.float32)] * 7
            + [pltpu.SemaphoreType.DMA, pltpu.SemaphoreType.DMA]
        ),
    )


TR = 256
GRID = NROW // TR


def _loss_body(pint, nint, ppop, npop, popp, popn, mkf, acc):
    i = pl.program_id(0)
    p_int = pint[...]
    n_int = nint[...]
    p_pop = ppop[...]
    n_pop = npop[...]
    mk = mkf[...]

    def logsig(x):
        return -(jnp.maximum(-x, 0.0) + jnp.log1p(jnp.exp(-jnp.abs(x))))

    t1 = mk * logsig(p_int - n_int)
    t2 = mk * logsig(n_pop - p_pop)
    ep = jnp.where(p_pop > 0, p_pop, jnp.exp(p_pop) - 1.0) + 1.0
    en = jnp.where(n_pop > 0, n_pop, jnp.exp(n_pop) - 1.0) + 1.0
    gp = jnp.exp(GAMMA * jnp.log(popp[...]))
    gn = jnp.exp(GAMMA * jnp.log(popn[...]))
    t3 = (1.0 - mk) * logsig(ep * gp - en * gn)
    t4 = logsig(p_int + p_pop - n_int - n_pop)
    part = jnp.stack([t1, t2, t3, t4])
    part = part.reshape(4, TR // 8, 8, 128).sum(axis=1)

    @pl.when(i == 0)
    def _():
        acc[...] = jnp.zeros_like(acc)

    acc[...] += part


def _make_loss():
    vec_spec = pl.BlockSpec((TR, 128), lambda i: (i, 0))
    acc_spec = pl.BlockSpec((4, 8, 128), lambda i: (0, 0, 0))
    return pl.pallas_call(
        _loss_body,
        grid=(GRID,),
        in_specs=[vec_spec] * 7,
        out_specs=acc_spec,
        out_shape=jax.ShapeDtypeStruct((4, 8, 128), jnp.float32),
    )


RED = 4096
WPT = NUM_E // NS
SROW = 32
TPT = N // NS
RPT = TPT // 128


def _dedup_body(idxp_h, idxn_h, idxu_h, sqp_h, sqn_h, squ_h, out_h,
                table, ibuf, sbuf, rbuf, obuf, ssem):
    cid = lax.axis_index("c")
    sid = lax.axis_index("s")

    def fill16(i, carry):
        rbuf[pl.ds(i * 16, 16)] = jnp.full((16,), -1.0, jnp.float32)
        return carry

    lax.fori_loop(0, RED // 16, fill16, 0)

    def memset(i, carry):
        off = pl.multiple_of(sid * WPT + i * RED, 8)
        pltpu.sync_copy(rbuf, table.at[pl.ds(off, RED)])
        return carry

    lax.fori_loop(0, WPT // RED, memset, 0)
    plsc.subcore_barrier()

    def scatter_from(idx_h, sq_h):
        def body(ci, carry):
            rb = sid * RPT + ci * SROW
            pltpu.sync_copy(idx_h.at[pl.ds(rb, SROW)], ibuf)
            pltpu.sync_copy(sq_h.at[pl.ds(rb, SROW)], sbuf)
            cps = [pltpu.async_copy(sbuf.at[r], table.at[ibuf.at[r]], ssem)
                   for r in range(SROW)]
            for cp in cps:
                cp.wait()
            return carry

        lax.fori_loop(0, RPT // SROW, body, 0)

    @pl.when(cid == 0)
    def _():
        scatter_from(idxp_h, sqp_h)
        scatter_from(idxn_h, sqn_h)

    @pl.when(cid == 1)
    def _():
        scatter_from(idxu_h, squ_h)

    plsc.subcore_barrier()

    def red_chunk(i, carry):
        off = pl.multiple_of(sid * WPT + i * RED, 8)
        pltpu.sync_copy(table.at[pl.ds(off, RED)], rbuf)

        def red16(v, c2):
            s2, n2 = c2
            for u in range(4):
                x = rbuf[pl.ds(v * 64 + u * 16, 16)]
                m = x >= 0.0
                s2 = s2 + jnp.where(m, x, 0.0)
                n2 = n2 + jnp.where(m, 1.0, 0.0)
            return (s2, n2)

        return lax.fori_loop(0, RED // 64, red16, carry)

    zero = jnp.zeros((16,), jnp.float32)
    ssum, scnt = lax.fori_loop(0, WPT // RED, red_chunk, (zero, zero))
    obuf[0, :] = ssum
    obuf[1, :] = scnt
    pltpu.sync_copy(obuf, out_h.at[cid, sid])


def _make_dedup():
    mesh = plsc.VectorSubcoreMesh(core_axis_name="c", subcore_axis_name="s",
                                  num_cores=NC, num_subcores=NS)
    return pl.kernel(
        _dedup_body,
        out_type=jax.ShapeDtypeStruct((NC, NS, 2, 16), jnp.float32),
        mesh=mesh,
        scratch_types=[
            pltpu.VMEM_SHARED((NUM_E,), jnp.float32),
            pltpu.VMEM((SROW, 128), jnp.int32),
            pltpu.VMEM((SROW, 128), jnp.float32),
            pltpu.VMEM((RED,), jnp.float32),
            pltpu.VMEM((2, 16), jnp.float32),
            pltpu.SemaphoreType.DMA,
        ],
    )


def kernel(user, item_p, item_n, mask, users_int_w, users_pop_w, items_int_w,
           items_pop_w, blen_pop):
    mkf_raw = mask.astype(jnp.float32)
    user2, itemp2, itemn2, mkf = _make_fmt()(user, item_p, item_n, mkf_raw)

    (p_int, n_int, p_pop, n_pop, popp, popn, squ, sqp, sqn) = (
        _make_gather()(user2, itemp2, itemn2, users_int_w,
                       users_pop_w, items_int_w, items_pop_w, blen_pop))

    acc = _make_loss()(p_int, n_int, p_pop, n_pop, popp, popn, mkf)

    dd = _make_dedup()(itemp2, itemn2, user2, sqp, sqn, squ)

    sums = jnp.sum(acc, axis=(1, 2))
    nf = jnp.float32(N)
    loss_int = -sums[0] / nf
    loss_pop = -sums[1] / nf - sums[2] / nf
    loss_total = -sums[3] / nf
    item_sum = jnp.sum(dd[0, :, 0, :])
    item_cnt = jnp.sum(dd[0, :, 1, :])
    user_sum = jnp.sum(dd[1, :, 0, :])
    user_cnt = jnp.sum(dd[1, :, 1, :])
    disc = item_sum / (item_cnt * EMB) + user_sum / (user_cnt * EMB)
    return (INT_W * loss_int + POP_W * loss_pop + loss_total
            - DIS_PEN * disc)
